# baseline (speedup 1.0000x reference)
import functools

import jax
import jax.numpy as jnp
from jax import lax
from jax.experimental import pallas as pl
from jax.experimental.pallas import tpu as pltpu
from jax.experimental.pallas import tpu_sc as plsc

U = 1024
DIM = 64
DEPTH = 16

BATCH = 4096
SEQ = 200
NC, NS = 2, 16
NW = NC * NS
HB = 2048
NBG = HB // 16
NTR = DIM // 8
NU = SEQ * NTR * (BATCH // HB) // NW
LAT = 6


def _embed_body(unique_ref, prim_ref, out_ref):
    u = unique_ref[:]
    m0 = prim_ref[0]
    m1 = prim_ref[1]
    mapsT = jnp.ones((DIM, U), jnp.float32)
    dn = (((1,), (0,)), ((), ()))
    for depth in range(DEPTH):
        shifted = u >> depth
        code = jnp.where(shifted > 1, shifted & 1, 2)
        a = lax.dot_general(m0, mapsT, dn, preferred_element_type=jnp.float32)
        b = lax.dot_general(m1, mapsT, dn, preferred_element_type=jnp.float32)
        mapsT = jnp.where(code == 0, a, jnp.where(code == 1, b, mapsT))
    out_ref[:] = mapsT


def _embed(unique, primitives):
    return pl.pallas_call(
        _embed_body,
        out_shape=jax.ShapeDtypeStruct((DIM, U), jnp.float32),
    )(unique.reshape(1, U), primitives)


def _gather_body(mapT_hbm, tableT_hbm, out_hbm,
                 table_v, ix0, ix1, st0, st1, is0, is1, os0, os1):
    wid = lax.axis_index("s") * NC + lax.axis_index("c")
    ixs = (ix0, ix1)
    sts = (st0, st1)
    iss = (is0, is1)
    oss = (os0, os1)

    pltpu.sync_copy(tableT_hbm, table_v)

    def unit_coords(g):
        uid = wid * NU + g
        s = uid // (NTR * 2)
        tr = (uid // 2) % NTR
        h = uid % 2
        return s, tr, h

    def fire_idx(ix, isem, g):
        s, _, h = unit_coords(g)
        pltpu.async_copy(mapT_hbm.at[s, pl.ds(h * HB, HB)], ix, isem)

    def wait_idx(ix, isem):
        pltpu.make_async_copy(mapT_hbm.at[0, pl.ds(0, HB)], ix, isem).wait()

    def compute(st, ix, g):
        _, tr, _ = unit_coords(g)
        dbase = tr * 8 * U
        for bg in range(NBG):
            idx = ix[pl.ds(bg * 16, 16)]
            idn = {0: idx + dbase}
            vals = {}
            for j in range(8 + LAT):
                if j < 8:
                    vals[j] = plsc.load_gather(table_v, [idn.pop(j)])
                    if j < 7:
                        idn[j + 1] = idx + (dbase + (j + 1) * U)
                if j >= LAT:
                    st[0, j - LAT, pl.ds(bg * 16, 16)] = vals.pop(j - LAT)

    def fire_out(st, osem, g):
        s, tr, h = unit_coords(g)
        pltpu.async_copy(
            st, out_hbm.at[pl.ds(s, 1), pl.ds(tr * 8, 8), pl.ds(h * HB, HB)],
            osem)

    def wait_out(st, osem):
        pltpu.make_async_copy(
            st, out_hbm.at[pl.ds(0, 1), pl.ds(0, 8), pl.ds(0, HB)],
            osem).wait()

    fire_idx(ix0, is0, 0)
    fire_idx(ix1, is1, 1)

    def pair(p, _):
        for k in range(2):
            g = 2 * p + k
            wait_idx(ixs[k], iss[k])

            @pl.when(p > 0)
            def _(k=k):
                wait_out(sts[k], oss[k])
            compute(sts[k], ixs[k], g)
            fire_out(sts[k], oss[k], g)

            @pl.when(g + 2 < NU)
            def _(k=k, g=g):
                fire_idx(ixs[k], iss[k], g + 2)
        return ()

    lax.fori_loop(0, NU // 2, pair, (), unroll=False)
    for k in range(2):
        wait_out(sts[k], oss[k])


@functools.partial(jax.jit, static_argnums=())
def _gather(mapT, tableT_flat):
    mesh = plsc.VectorSubcoreMesh(core_axis_name="c", subcore_axis_name="s")
    f = pl.kernel(
        _gather_body,
        out_type=jax.ShapeDtypeStruct((SEQ, DIM, BATCH), jnp.float32),
        mesh=mesh,
        scratch_types=[
            pltpu.VMEM((DIM * U,), jnp.float32),
            pltpu.VMEM((HB,), jnp.int32),
            pltpu.VMEM((HB,), jnp.int32),
            pltpu.VMEM((1, 8, HB), jnp.float32),
            pltpu.VMEM((1, 8, HB), jnp.float32),
            pltpu.SemaphoreType.DMA,
            pltpu.SemaphoreType.DMA,
            pltpu.SemaphoreType.DMA,
            pltpu.SemaphoreType.DMA,
        ],
        compiler_params=pltpu.CompilerParams(
            use_tc_tiling_on_sc=True, needs_layout_passes=False),
    )
    return f(mapT, tableT_flat)


def kernel(unique, mapping, primitives):
    tableT = _embed(unique, primitives)
    outP = _gather(mapping.T, tableT.reshape(DIM * U))
    return jnp.transpose(outP, (2, 0, 1))


# speedup vs baseline: 1.6787x; 1.6787x over previous
import functools

import jax
import jax.numpy as jnp
from jax import lax
from jax.experimental import pallas as pl
from jax.experimental.pallas import tpu as pltpu
from jax.experimental.pallas import tpu_sc as plsc

U = 1024
DIM = 64
DEPTH = 16

BATCH = 4096
SEQ = 200
NC, NS = 2, 16
NW = NC * NS
LW = 128
NBG = LW // 16
NPAIR = SEQ // 2


def _embed_body(unique_ref, prim_ref, out_ref):
    u = unique_ref[:]
    m0 = prim_ref[0]
    m1 = prim_ref[1]
    mapsT = jnp.ones((DIM, U), jnp.float32)
    dn = (((1,), (0,)), ((), ()))
    for depth in range(DEPTH):
        shifted = u >> depth
        code = jnp.where(shifted > 1, shifted & 1, 2)
        a = lax.dot_general(m0, mapsT, dn, preferred_element_type=jnp.float32)
        b = lax.dot_general(m1, mapsT, dn, preferred_element_type=jnp.float32)
        mapsT = jnp.where(code == 0, a, jnp.where(code == 1, b, mapsT))
    out_ref[:] = mapsT


def _embed(unique, primitives):
    return pl.pallas_call(
        _embed_body,
        out_shape=jax.ShapeDtypeStruct((DIM, U), jnp.float32),
    )(unique.reshape(1, U), primitives)


def _gather_body(mapT_hbm, tableT_hbm, out_hbm,
                 table_v, idx_v, st0, st1, st2, st3, os0, os1, os2, os3):
    wid = lax.axis_index("s") * NC + lax.axis_index("c")
    lane0 = wid * LW
    sts = (st0, st1)
    oss = (os0, os1)
    del st2, st3, os2, os3

    pltpu.sync_copy(tableT_hbm, table_v)
    pltpu.sync_copy(mapT_hbm.at[:, pl.ds(lane0, LW)], idx_v)

    LAT = 6

    def compute(st, s):
        idxs = {0: idx_v[s, pl.ds(0, 16)]}
        for bg in range(NBG):
            idx = idxs.pop(bg)
            vals = {}
            for d in range(DIM + LAT):
                if d < DIM:
                    vals[d] = plsc.load_gather(
                        table_v.at[pl.ds(d * U, U)], [idx])
                    if d == 1 and bg + 1 < NBG:
                        idxs[bg + 1] = idx_v[s, pl.ds((bg + 1) * 16, 16)]
                if d >= LAT:
                    st[0, d - LAT, pl.ds(bg * 16, 16)] = vals.pop(d - LAT)

    def fire_out(st, osem, s):
        pltpu.async_copy(
            st, out_hbm.at[pl.ds(s, 1), :, pl.ds(lane0, LW)], osem)

    def wait_out(st, osem):
        pltpu.make_async_copy(
            st, out_hbm.at[pl.ds(0, 1), :, pl.ds(lane0, LW)], osem).wait()

    def pair(p, _):
        s0 = 2 * p
        for k in range(2):
            @pl.when(p > 0)
            def _(k=k):
                wait_out(sts[k], oss[k])
            compute(sts[k], s0 + k)
            fire_out(sts[k], oss[k], s0 + k)
        return ()

    lax.fori_loop(0, NPAIR, pair, (), unroll=False)
    for k in range(2):
        wait_out(sts[k], oss[k])


@functools.partial(jax.jit, static_argnums=())
def _gather(mapT, tableT_flat):
    mesh = plsc.VectorSubcoreMesh(core_axis_name="c", subcore_axis_name="s")
    f = pl.kernel(
        _gather_body,
        out_type=jax.ShapeDtypeStruct((SEQ, DIM, BATCH), jnp.float32),
        mesh=mesh,
        scratch_types=[
            pltpu.VMEM((DIM * U,), jnp.float32),
            pltpu.VMEM((SEQ, LW), jnp.int32),
            pltpu.VMEM((1, DIM, LW), jnp.float32),
            pltpu.VMEM((1, DIM, LW), jnp.float32),
            pltpu.VMEM((1, DIM, LW), jnp.float32),
            pltpu.VMEM((1, DIM, LW), jnp.float32),
            pltpu.SemaphoreType.DMA,
            pltpu.SemaphoreType.DMA,
            pltpu.SemaphoreType.DMA,
            pltpu.SemaphoreType.DMA,
        ],
        compiler_params=pltpu.CompilerParams(
            use_tc_tiling_on_sc=True, needs_layout_passes=False),
    )
    return f(mapT, tableT_flat)


def kernel(unique, mapping, primitives):
    tableT = _embed(unique, primitives)
    outP = _gather(mapping.T, tableT.reshape(DIM * U))
    return jnp.transpose(outP, (2, 0, 1))


# speedup vs baseline: 2.5414x; 1.5140x over previous
import functools

import jax
import jax.numpy as jnp
from jax import lax
from jax.experimental import pallas as pl
from jax.experimental.pallas import tpu as pltpu
from jax.experimental.pallas import tpu_sc as plsc

U = 1024
DIM = 64
DEPTH = 16

BATCH = 4096
SEQ = 200
NC, NS = 2, 16
NW = NC * NS
LW = 128
NBG = LW // 16
NPAIR = SEQ // 2


def _embed_body(unique_ref, prim_ref, out_ref):
    u = unique_ref[:]
    m0 = prim_ref[0]
    m1 = prim_ref[1]
    mapsT = jnp.ones((DIM, U), jnp.float32)
    dn = (((1,), (0,)), ((), ()))
    for depth in range(DEPTH):
        shifted = u >> depth
        code = jnp.where(shifted > 1, shifted & 1, 2)
        a = lax.dot_general(m0, mapsT, dn, preferred_element_type=jnp.float32)
        b = lax.dot_general(m1, mapsT, dn, preferred_element_type=jnp.float32)
        mapsT = jnp.where(code == 0, a, jnp.where(code == 1, b, mapsT))
    out_ref[:] = mapsT


def _embed(unique, primitives):
    return pl.pallas_call(
        _embed_body,
        out_shape=jax.ShapeDtypeStruct((DIM, U), jnp.float32),
    )(unique.reshape(1, U), primitives)


def _gather_body(mapT_hbm, tableT_hbm, out_hbm,
                 table_v, idx_v, st0, st1, st2, st3, os0, os1, os2, os3):
    wid = lax.axis_index("s") * NC + lax.axis_index("c")
    lane0 = wid * LW
    sts = (st0, st1)
    oss = (os0, os1)
    del st2, st3, os2, os3

    pltpu.sync_copy(tableT_hbm, table_v)
    pltpu.sync_copy(mapT_hbm.at[:, pl.ds(lane0, LW)], idx_v)

    LAT = 6
    HD = DIM // 2

    def compute_half(st, idxs, h):
        items = [(d, bg) for d in range(h * HD, h * HD + HD)
                 for bg in range(NBG)]
        vals = {}
        for i in range(len(items) + LAT):
            if i < len(items):
                d, bg = items[i]
                vals[i] = plsc.load_gather(
                    table_v.at[pl.ds(d * U, U)], [idxs[bg]])
            if i >= LAT:
                d, bg = items[i - LAT]
                st[0, d, pl.ds(bg * 16, 16)] = vals.pop(i - LAT)

    def fire_half(st, osem, s, h):
        pltpu.async_copy(
            st.at[:, pl.ds(h * HD, HD), :],
            out_hbm.at[pl.ds(s, 1), pl.ds(h * HD, HD), pl.ds(lane0, LW)],
            osem)

    def wait_half(st, osem, h):
        pltpu.make_async_copy(
            st.at[:, pl.ds(h * HD, HD), :],
            out_hbm.at[pl.ds(0, 1), pl.ds(h * HD, HD), pl.ds(lane0, LW)],
            osem).wait()

    def pair(p, _):
        s0 = 2 * p
        for k in range(2):
            s = s0 + k
            idxs = [idx_v[s, pl.ds(bg * 16, 16)] for bg in range(NBG)]
            for h in range(2):
                @pl.when(p > 0)
                def _(k=k, h=h):
                    wait_half(sts[k], oss[k], h)
                compute_half(sts[k], idxs, h)
                fire_half(sts[k], oss[k], s, h)
        return ()

    lax.fori_loop(0, NPAIR, pair, (), unroll=False)
    for k in range(2):
        for h in range(2):
            wait_half(sts[k], oss[k], h)


@functools.partial(jax.jit, static_argnums=())
def _gather(mapT, tableT_flat):
    mesh = plsc.VectorSubcoreMesh(core_axis_name="c", subcore_axis_name="s")
    f = pl.kernel(
        _gather_body,
        out_type=jax.ShapeDtypeStruct((SEQ, DIM, BATCH), jnp.float32),
        mesh=mesh,
        scratch_types=[
            pltpu.VMEM((DIM * U,), jnp.float32),
            pltpu.VMEM((SEQ, LW), jnp.int32),
            pltpu.VMEM((1, DIM, LW), jnp.float32),
            pltpu.VMEM((1, DIM, LW), jnp.float32),
            pltpu.VMEM((1, DIM, LW), jnp.float32),
            pltpu.VMEM((1, DIM, LW), jnp.float32),
            pltpu.SemaphoreType.DMA,
            pltpu.SemaphoreType.DMA,
            pltpu.SemaphoreType.DMA,
            pltpu.SemaphoreType.DMA,
        ],
        compiler_params=pltpu.CompilerParams(
            use_tc_tiling_on_sc=True, needs_layout_passes=False),
    )
    return f(mapT, tableT_flat)


def kernel(unique, mapping, primitives):
    tableT = _embed(unique, primitives)
    outP = _gather(mapping.T, tableT.reshape(DIM * U))
    return jnp.transpose(outP, (2, 0, 1))


# speedup vs baseline: 2.5793x; 1.0149x over previous
import functools

import jax
import jax.numpy as jnp
from jax import lax
from jax.experimental import pallas as pl
from jax.experimental.pallas import tpu as pltpu
from jax.experimental.pallas import tpu_sc as plsc

U = 1024
DIM = 64
DEPTH = 16

BATCH = 4096
SEQ = 200
NC, NS = 2, 16
NW = NC * NS
LW = 128
NBG = LW // 16
NPAIR = SEQ // 2


def _embed_body(unique_ref, prim_ref, out_ref):
    u = unique_ref[:]
    m0 = prim_ref[0]
    m1 = prim_ref[1]
    mapsT = jnp.ones((DIM, U), jnp.float32)
    dn = (((1,), (0,)), ((), ()))
    for depth in range(DEPTH):
        shifted = u >> depth
        code = jnp.where(shifted > 1, shifted & 1, 2)
        a = lax.dot_general(m0, mapsT, dn, preferred_element_type=jnp.float32)
        b = lax.dot_general(m1, mapsT, dn, preferred_element_type=jnp.float32)
        mapsT = jnp.where(code == 0, a, jnp.where(code == 1, b, mapsT))
    out_ref[:] = mapsT


def _embed(unique, primitives):
    return pl.pallas_call(
        _embed_body,
        out_shape=jax.ShapeDtypeStruct((DIM, U), jnp.float32),
    )(unique.reshape(1, U), primitives)


def _gather_body(mapT_hbm, tableT_hbm, out_hbm,
                 table_v, idx_v, st0, st1, st2, st3, os0, os1, os2, os3):
    wid = lax.axis_index("s") * NC + lax.axis_index("c")
    lane0 = wid * LW
    sts = (st0, st1)
    oss = (os0, os1)
    del st2, st3, os2, os3

    pltpu.sync_copy(tableT_hbm, table_v)
    pltpu.sync_copy(mapT_hbm.at[:, pl.ds(lane0, LW)], idx_v)

    LAT = 6
    NH = 4
    HD = DIM // NH

    def compute_half(st, idxs, h):
        items = [(d, bg) for d in range(h * HD, h * HD + HD)
                 for bg in range(NBG)]
        vals = {}
        for i in range(len(items) + LAT):
            if i < len(items):
                d, bg = items[i]
                vals[i] = plsc.load_gather(
                    table_v.at[pl.ds(d * U, U)], [idxs[bg]])
            if i >= LAT:
                d, bg = items[i - LAT]
                st[0, d, pl.ds(bg * 16, 16)] = vals.pop(i - LAT)

    def fire_half(st, osem, s, h):
        pltpu.async_copy(
            st.at[:, pl.ds(h * HD, HD), :],
            out_hbm.at[pl.ds(s, 1), pl.ds(h * HD, HD), pl.ds(lane0, LW)],
            osem)

    def wait_half(st, osem, h):
        pltpu.make_async_copy(
            st.at[:, pl.ds(h * HD, HD), :],
            out_hbm.at[pl.ds(0, 1), pl.ds(h * HD, HD), pl.ds(lane0, LW)],
            osem).wait()

    def pair(p, _):
        s0 = 2 * p
        for k in range(2):
            s = s0 + k
            idxs = [idx_v[s, pl.ds(bg * 16, 16)] for bg in range(NBG)]
            for h in range(NH):
                @pl.when(p > 0)
                def _(k=k, h=h):
                    wait_half(sts[k], oss[k], h)
                compute_half(sts[k], idxs, h)
                fire_half(sts[k], oss[k], s, h)
        return ()

    lax.fori_loop(0, NPAIR, pair, (), unroll=False)
    for k in range(2):
        for h in range(NH):
            wait_half(sts[k], oss[k], h)


@functools.partial(jax.jit, static_argnums=())
def _gather(mapT, tableT_flat):
    mesh = plsc.VectorSubcoreMesh(core_axis_name="c", subcore_axis_name="s")
    f = pl.kernel(
        _gather_body,
        out_type=jax.ShapeDtypeStruct((SEQ, DIM, BATCH), jnp.float32),
        mesh=mesh,
        scratch_types=[
            pltpu.VMEM((DIM * U,), jnp.float32),
            pltpu.VMEM((SEQ, LW), jnp.int32),
            pltpu.VMEM((1, DIM, LW), jnp.float32),
            pltpu.VMEM((1, DIM, LW), jnp.float32),
            pltpu.VMEM((1, DIM, LW), jnp.float32),
            pltpu.VMEM((1, DIM, LW), jnp.float32),
            pltpu.SemaphoreType.DMA,
            pltpu.SemaphoreType.DMA,
            pltpu.SemaphoreType.DMA,
            pltpu.SemaphoreType.DMA,
        ],
        compiler_params=pltpu.CompilerParams(
            use_tc_tiling_on_sc=True, needs_layout_passes=False),
    )
    return f(mapT, tableT_flat)


def kernel(unique, mapping, primitives):
    tableT = _embed(unique, primitives)
    outP = _gather(mapping.T, tableT.reshape(DIM * U))
    return jnp.transpose(outP, (2, 0, 1))
